# baseline (speedup 1.0000x reference)
import functools

import jax
import jax.numpy as jnp
from jax import lax
from jax.experimental import pallas as pl
from jax.experimental.pallas import tpu as pltpu
from jax.experimental.pallas import tpu_sc as plsc

_N = 10000
_M = 10000
_P = 320000
_C = 128
_K = 64
_DEPTH = 2
_HEADS = 4
_NEG = 0.2

_NC = 2
_NS = 16
_NW = _NC * _NS
_CHUNK = 80
_PPT = _P // _NW
_NCHUNK = _PPT // _CHUNK
_SCH = 5
_NSUP = _NCHUNK // _SCH

_HIGH = lax.Precision.HIGHEST

_SC_PARAMS = pltpu.CompilerParams(use_tc_tiling_on_sc=False,
                                  needs_layout_passes=False)

_mesh = plsc.VectorSubcoreMesh(core_axis_name="c", subcore_axis_name="s",
                               num_cores=_NC, num_subcores=_NS)


def _lin_body(cout, r, nh, h_ref, w_ref, b_ref, ad_ref, *outs):
    pad1 = jnp.where(
        lax.broadcasted_iota(jnp.int32, (h_ref.shape[0], r - cout), 1) == 0,
        1.0, 0.0)
    for hd in range(nh):
        xl = jnp.dot(h_ref[...], w_ref[hd], precision=_HIGH,
                     preferred_element_type=jnp.float32) + b_ref[hd]
        outs[2 * hd][:, :cout] = xl
        outs[2 * hd][:, cout:] = pad1
        outs[2 * hd + 1][...] = jnp.dot(xl, ad_ref[hd], precision=_HIGH,
                                        preferred_element_type=jnp.float32)


def _tc_linear(h, w4, b4, ad4, cout, r):
    n, cin = h.shape
    nh = w4.shape[0]
    bn = 400
    out_shape = []
    out_specs = []
    for _ in range(nh):
        out_shape += [jax.ShapeDtypeStruct((n, r), jnp.float32),
                      jax.ShapeDtypeStruct((n, 1), jnp.float32)]
        out_specs += [pl.BlockSpec((bn, r), lambda i: (i, 0)),
                      pl.BlockSpec((bn, 1), lambda i: (i, 0))]
    return pl.pallas_call(
        functools.partial(_lin_body, cout, r, nh),
        grid=(n // bn,),
        in_specs=[
            pl.BlockSpec((bn, cin), lambda i: (i, 0)),
            pl.BlockSpec((nh, cin, cout), lambda i: (0, 0, 0)),
            pl.BlockSpec((nh, 1, cout), lambda i: (0, 0, 0)),
            pl.BlockSpec((nh, cout, 1), lambda i: (0, 0, 0)),
        ],
        out_specs=out_specs,
        out_shape=out_shape,
    )(h, w4.astype(jnp.float32), b4.reshape(nh, 1, cout).astype(jnp.float32),
      ad4.reshape(nh, cout, 1).astype(jnp.float32))


def _edge_body(cout, r, nh, *refs):
    ae_ref = refs[nh]
    lane = None
    for hd in range(nh):
        s = refs[hd][0] + refs[hd][1]
        deg = s[:, cout:cout + 1]
        inv = 1.0 / jnp.maximum(deg, 1.0)
        y = s[:, :cout] * inv
        ypad_ref = refs[nh + 1 + hd]
        ypad_ref[:, :cout] = y
        alpha = jnp.dot(y, ae_ref[hd], precision=_HIGH,
                        preferred_element_type=jnp.float32)
        if lane is None:
            lane = lax.broadcasted_iota(jnp.int32,
                                        (y.shape[0], r - cout), 1)
        ypad_ref[:, cout:] = jnp.where(
            lane == 0, 1.0, jnp.where(lane == 1, alpha, 0.0))


def _tc_edge(yparts_list, ae4, cout, r):
    nh = len(yparts_list)
    bm = 400
    yps = [yp.reshape(2, _M, r) for yp in yparts_list]
    return pl.pallas_call(
        functools.partial(_edge_body, cout, r, nh),
        grid=(_M // bm,),
        in_specs=[pl.BlockSpec((2, bm, r), lambda i: (0, i, 0))
                  for _ in range(nh)] +
                 [pl.BlockSpec((nh, cout, 1), lambda i: (0, 0, 0))],
        out_specs=[pl.BlockSpec((bm, r), lambda i: (i, 0))
                   for _ in range(nh)],
        out_shape=[jax.ShapeDtypeStruct((_M, r), jnp.float32)
                   for _ in range(nh)],
    )(*yps, ae4.reshape(nh, cout, 1).astype(jnp.float32))


def _vert_body(cout, r, nh, *refs):
    out_ref = refs[-1]
    acc = None
    for xr in refs[:-1]:
        s = xr[0] + xr[1]
        den = s[:, cout:cout + 1] + 1e-12
        xo = s[:, :cout] / den
        e = jnp.where(xo > 0, xo, jnp.exp(jnp.minimum(xo, 0.0)) - 1.0)
        acc = e if acc is None else acc + e
    out_ref[...] = acc * (1.0 / nh)


def _tc_vertex(xoparts, cout, r):
    nh = len(xoparts)
    bn = 400
    xps = [x.reshape(2, _N, r) for x in xoparts]
    return pl.pallas_call(
        functools.partial(_vert_body, cout, r, nh),
        grid=(_N // bn,),
        in_specs=[pl.BlockSpec((2, bn, r), lambda i: (0, i, 0))
                  for _ in range(nh)],
        out_specs=pl.BlockSpec((bn, cout), lambda i: (i, 0)),
        out_shape=jax.ShapeDtypeStruct((_N, cout), jnp.float32),
    )(*xps)


_NZCHUNK = _M // _CHUNK
_ZPT = -(-_NZCHUNK // _NS)


def _zero_rowbuf(row_buf, ncg):
    @pl.loop(0, _CHUNK)
    def _(i):
        for g in range(ncg):
            row_buf[i, pl.ds(g * 16, 16)] = jnp.zeros((16,), jnp.float32)


def _zero_and_barrier(row_buf, acc_sh, s, ncg, sem):
    _zero_rowbuf(row_buf, ncg)
    for k in range(_ZPT):
        j = s + k * _NS

        @pl.when(j < _NZCHUNK)
        def _():
            pltpu.async_copy(row_buf, acc_sh.at[pl.ds(j * _CHUNK, _CHUNK)],
                             sem)
    for k in range(_ZPT):
        j = s + k * _NS

        @pl.when(j < _NZCHUNK)
        def _():
            pltpu.make_async_copy(
                row_buf, acc_sh.at[pl.ds(j * _CHUNK, _CHUNK)], sem).wait()
    plsc.subcore_barrier()


def _writeout(acc_sh, out_hbm, c, s, sem):
    plsc.subcore_barrier()
    for k in range(_ZPT):
        j = s + k * _NS

        @pl.when(j < _NZCHUNK)
        def _():
            off = j * _CHUNK
            pltpu.async_copy(acc_sh.at[pl.ds(off, _CHUNK)],
                             out_hbm.at[pl.ds(c * _M + off, _CHUNK)], sem)
    for k in range(_ZPT):
        j = s + k * _NS

        @pl.when(j < _NZCHUNK)
        def _():
            off = j * _CHUNK
            pltpu.make_async_copy(
                acc_sh.at[pl.ds(off, _CHUNK)],
                out_hbm.at[pl.ds(c * _M + off, _CHUNK)], sem).wait()


def _fetch_idx(pvpe_hbm, wid, sup, idx_buf, sem_i):
    pltpu.async_copy(pvpe_hbm.at[wid, pl.ds(sup * _SCH, _SCH)], idx_buf,
                     sem_i)


def _wait_idx(pvpe_hbm, wid, idx_buf, sem_i):
    pltpu.make_async_copy(pvpe_hbm.at[wid, pl.ds(0, _SCH)], idx_buf,
                          sem_i).wait()


def _v2e_sup(first, gather_hbm, cur, prev, row_bufs, acc_sh,
             sems, sem_g, prefetch):
    gd = {}
    for i in range(_SCH):
        b = i % 3
        if i < 3:
            if not first:
                pltpu.make_async_copy(
                    row_bufs.at[b], acc_sh.at[prev.at[_SCH - 3 + i, 1]],
                    sems[b]).wait()
        else:
            pltpu.make_async_copy(
                row_bufs.at[b], acc_sh.at[cur.at[i - 3, 1]], sems[b]).wait()
        gd[i] = pltpu.async_copy(gather_hbm.at[cur.at[i, 0]],
                                 row_bufs.at[b], sem_g)
        if i == 2 and prefetch is not None:
            prefetch()
        if i >= 1:
            gd[i - 1].wait()
            pltpu.async_copy(row_bufs.at[(i - 1) % 3],
                             acc_sh.at[cur.at[i - 1, 1]],
                             sems[(i - 1) % 3], add=True)
    gd[_SCH - 1].wait()
    pltpu.async_copy(row_bufs.at[(_SCH - 1) % 3],
                     acc_sh.at[cur.at[_SCH - 1, 1]],
                     sems[(_SCH - 1) % 3], add=True)


def _v2e_body(r, xlp_hbm, pvpe_hbm, out_hbm,
              idx_a, idx_b, row_bufs, acc_sh, sem_i, sem_g, s0, s1, s2):
    c = lax.axis_index("c")
    s = lax.axis_index("s")
    wid = c * _NS + s
    sems = (s0, s1, s2)
    _fetch_idx(pvpe_hbm, wid, 0, idx_a, sem_i)
    _zero_and_barrier(row_bufs.at[0], acc_sh, s, r // 16, sem_g)

    _wait_idx(pvpe_hbm, wid, idx_a, sem_i)
    _v2e_sup(True, xlp_hbm, idx_a, idx_b, row_bufs, acc_sh, sems, sem_g,
             lambda: _fetch_idx(pvpe_hbm, wid, 1, idx_b, sem_i))

    npair = (_NSUP - 1) // 2

    @pl.loop(0, npair)
    def _(t):
        _wait_idx(pvpe_hbm, wid, idx_b, sem_i)
        _v2e_sup(False, xlp_hbm, idx_b, idx_a, row_bufs, acc_sh,
                 sems, sem_g,
                 lambda: _fetch_idx(pvpe_hbm, wid, 2 * t + 2, idx_a, sem_i))
        _wait_idx(pvpe_hbm, wid, idx_a, sem_i)

        def pf():
            @pl.when(t < npair - 1)
            def _():
                _fetch_idx(pvpe_hbm, wid, 2 * t + 3, idx_b, sem_i)
        _v2e_sup(False, xlp_hbm, idx_a, idx_b, row_bufs, acc_sh,
                 sems, sem_g, pf)

    for i in range(3):
        b = (_SCH - 3 + i) % 3
        pltpu.make_async_copy(row_bufs.at[b],
                              acc_sh.at[idx_a.at[_SCH - 3 + i, 1]],
                              sems[b]).wait()

    _writeout(acc_sh, out_hbm, c, s, sem_g)


def _sc_v2e(xl_pad, pvpe, r):
    k = pl.kernel(
        functools.partial(_v2e_body, r),
        out_type=jax.ShapeDtypeStruct((_NC * _M, r), jnp.float32),
        mesh=_mesh,
        scratch_types=[
            pltpu.VMEM((_SCH, 2, _CHUNK), jnp.int32),
            pltpu.VMEM((_SCH, 2, _CHUNK), jnp.int32),
            pltpu.VMEM((3, _CHUNK, r), jnp.float32),
            pltpu.VMEM_SHARED((_M, r), jnp.float32),
            pltpu.SemaphoreType.DMA,
            pltpu.SemaphoreType.DMA,
            pltpu.SemaphoreType.DMA,
            pltpu.SemaphoreType.DMA,
            pltpu.SemaphoreType.DMA,
        ],
        compiler_params=_SC_PARAMS,
    )
    return k(xl_pad, pvpe)


def _e2v_compute(b, cout, ncg, idx, i, row_bufs, ex_buf, xd_buf):
    @pl.loop(0, _CHUNK // 16)
    def _(g):
        g16 = g * 16 + lax.iota(jnp.int32, 16)
        al = plsc.load_gather(row_bufs.at[b],
                              [g16, jnp.full((16,), cout + 1, jnp.int32)])
        pvv = idx[i, 0, pl.ds(g * 16, 16)]
        xd = plsc.load_gather(xd_buf, [pvv])
        sc = al + xd
        sc = jnp.where(sc >= 0, sc, _NEG * sc)
        ex_buf[pl.ds(g * 16, 16)] = jnp.exp(sc)

    @pl.loop(0, _CHUNK, unroll=8)
    def _(rr):
        exi = plsc.load_gather(ex_buf, [jnp.zeros((16,), jnp.int32) + rr])
        for g in range(ncg):
            row_bufs[b, rr, pl.ds(g * 16, 16)] = (
                row_bufs[b, rr, pl.ds(g * 16, 16)] * exi)


def _e2v_sup(first, cout, ncg, ypad_hbm, cur, prev, row_bufs, ex_buf, xd_buf,
             acc_sh, sems, sem_g, prefetch):
    gd = {}

    def process(i):
        b = i % 2
        gd[i].wait()
        _e2v_compute(b, cout, ncg, cur, i, row_bufs, ex_buf, xd_buf)
        pltpu.async_copy(row_bufs.at[b], acc_sh.at[cur.at[i, 0]],
                         sems[b], add=True)

    for i in range(_SCH):
        b = i % 2
        if i < 2:
            if not first:
                pltpu.make_async_copy(
                    row_bufs.at[b], acc_sh.at[prev.at[_SCH - 2 + i, 0]],
                    sems[b]).wait()
        else:
            pltpu.make_async_copy(
                row_bufs.at[b], acc_sh.at[cur.at[i - 2, 0]], sems[b]).wait()
        gd[i] = pltpu.async_copy(ypad_hbm.at[cur.at[i, 1]],
                                 row_bufs.at[b], sem_g)
        if i == 1 and prefetch is not None:
            prefetch()
        if i >= 1:
            process(i - 1)
    process(_SCH - 1)


def _e2v_body(cout, r, ypad_hbm, xd_hbm, pvpe_hbm, out_hbm,
              idx_a, idx_b, row_bufs, ex_buf, xd_buf, acc_sh,
              sem_i, sem_g, s0, s1):
    c = lax.axis_index("c")
    s = lax.axis_index("s")
    wid = c * _NS + s
    ncg = r // 16
    sems = (s0, s1)
    _fetch_idx(pvpe_hbm, wid, 0, idx_a, sem_i)
    pltpu.sync_copy(xd_hbm, xd_buf)
    _zero_and_barrier(row_bufs.at[0], acc_sh, s, ncg, sem_g)

    _wait_idx(pvpe_hbm, wid, idx_a, sem_i)
    _e2v_sup(True, cout, ncg, ypad_hbm, idx_a, idx_b, row_bufs, ex_buf,
             xd_buf, acc_sh, sems, sem_g,
             lambda: _fetch_idx(pvpe_hbm, wid, 1, idx_b, sem_i))

    npair = (_NSUP - 1) // 2

    @pl.loop(0, npair)
    def _(t):
        _wait_idx(pvpe_hbm, wid, idx_b, sem_i)
        _e2v_sup(False, cout, ncg, ypad_hbm, idx_b, idx_a, row_bufs, ex_buf,
                 xd_buf, acc_sh, sems, sem_g,
                 lambda: _fetch_idx(pvpe_hbm, wid, 2 * t + 2, idx_a, sem_i))
        _wait_idx(pvpe_hbm, wid, idx_a, sem_i)

        def pf():
            @pl.when(t < npair - 1)
            def _():
                _fetch_idx(pvpe_hbm, wid, 2 * t + 3, idx_b, sem_i)
        _e2v_sup(False, cout, ncg, ypad_hbm, idx_a, idx_b, row_bufs, ex_buf,
                 xd_buf, acc_sh, sems, sem_g, pf)

    for i in range(2):
        b = (_SCH - 2 + i) % 2
        pltpu.make_async_copy(row_bufs.at[b],
                              acc_sh.at[idx_a.at[_SCH - 2 + i, 0]],
                              sems[b]).wait()

    _writeout(acc_sh, out_hbm, c, s, sem_g)


def _sc_e2v(y_pad, xdst, pvpe, cout, r):
    k = pl.kernel(
        functools.partial(_e2v_body, cout, r),
        out_type=jax.ShapeDtypeStruct((_NC * _N, r), jnp.float32),
        mesh=_mesh,
        scratch_types=[
            pltpu.VMEM((_SCH, 2, _CHUNK), jnp.int32),
            pltpu.VMEM((_SCH, 2, _CHUNK), jnp.int32),
            pltpu.VMEM((2, _CHUNK, r), jnp.float32),
            pltpu.VMEM((_CHUNK,), jnp.float32),
            pltpu.VMEM((_N,), jnp.float32),
            pltpu.VMEM_SHARED((_N, r), jnp.float32),
            pltpu.SemaphoreType.DMA,
            pltpu.SemaphoreType.DMA,
            pltpu.SemaphoreType.DMA,
            pltpu.SemaphoreType.DMA,
        ],
        compiler_params=_SC_PARAMS,
    )
    return k(y_pad, xdst, pvpe)


def _conv_layer(h, pvpe, w4, b4, ae4, ad4, cout):
    r = cout + 16
    nh = w4.shape[0]
    lin_outs = _tc_linear(h, w4, b4, ad4, cout, r)
    xl_pads = lin_outs[0::2]
    xdsts = lin_outs[1::2]
    yparts = [_sc_v2e(xl_pads[hd], pvpe, r) for hd in range(nh)]
    y_pads = _tc_edge(yparts, ae4, cout, r)
    return [_sc_e2v(y_pads[hd], xdsts[hd].reshape(_N), pvpe, cout, r)
            for hd in range(nh)]


def kernel(X, pair_v, pair_e, W_layers, b_layers, ae_layers, ad_layers,
           W_out, b_out, ae_out, ad_out):
    h = X.astype(jnp.float32)
    pv3 = pair_v.astype(jnp.int32).reshape(_NW, _NCHUNK, 1, _CHUNK)
    pe3 = pair_e.astype(jnp.int32).reshape(_NW, _NCHUNK, 1, _CHUNK)
    pvpe = jnp.concatenate([pv3, pe3], axis=2)
    for l in range(_DEPTH):
        parts = _conv_layer(h, pvpe, W_layers[l], b_layers[l],
                            ae_layers[l], ad_layers[l], _C)
        h = _tc_vertex(parts, _C, _C + 16)
    xop = _conv_layer(h, pvpe, W_out.reshape(1, _C, _K),
                      b_out.reshape(1, _K), ae_out.reshape(1, _K),
                      ad_out.reshape(1, _K), _K)
    return _tc_vertex(xop, _K, _K + 16)


# speedup vs baseline: 1.6769x; 1.6769x over previous
import functools

import jax
import jax.numpy as jnp
from jax import lax
from jax.experimental import pallas as pl
from jax.experimental.pallas import tpu as pltpu
from jax.experimental.pallas import tpu_sc as plsc

_N = 10000
_M = 10000
_P = 320000
_C = 128
_K = 64
_DEPTH = 2
_HEADS = 4
_NEG = 0.2

_NC = 2
_NS = 16
_NW = _NC * _NS
_CHUNK = 80
_PPT = _P // _NW
_NCHUNK = _PPT // _CHUNK
_SCH = 5
_NSUP = _NCHUNK // _SCH

_HIGH = lax.Precision.HIGHEST

_SC_PARAMS = pltpu.CompilerParams(use_tc_tiling_on_sc=False,
                                  needs_layout_passes=False)

_mesh = plsc.VectorSubcoreMesh(core_axis_name="c", subcore_axis_name="s",
                               num_cores=_NC, num_subcores=_NS)


def _lin_body(cout, r, h_ref, w_ref, b_ref, ad_ref, xlp_ref, xd_ref):
    xl = jnp.dot(h_ref[...], w_ref[...], precision=_HIGH,
                 preferred_element_type=jnp.float32) + b_ref[...]
    xlp_ref[:, :cout] = xl
    pad = jnp.where(
        lax.broadcasted_iota(jnp.int32, (xl.shape[0], r - cout), 1) == 0,
        1.0, 0.0)
    xlp_ref[:, cout:] = pad
    xd_ref[...] = jnp.dot(xl, ad_ref[...], precision=_HIGH,
                          preferred_element_type=jnp.float32)


def _tc_linear(h, w, b, ad, cout, r):
    n, cin = h.shape
    bn = 400
    return pl.pallas_call(
        functools.partial(_lin_body, cout, r),
        grid=(n // bn,),
        in_specs=[
            pl.BlockSpec((bn, cin), lambda i: (i, 0)),
            pl.BlockSpec((cin, cout), lambda i: (0, 0)),
            pl.BlockSpec((1, cout), lambda i: (0, 0)),
            pl.BlockSpec((cout, 1), lambda i: (0, 0)),
        ],
        out_specs=[
            pl.BlockSpec((bn, r), lambda i: (i, 0)),
            pl.BlockSpec((bn, 1), lambda i: (i, 0)),
        ],
        out_shape=[
            jax.ShapeDtypeStruct((n, r), jnp.float32),
            jax.ShapeDtypeStruct((n, 1), jnp.float32),
        ],
    )(h, w.astype(jnp.float32), b.reshape(1, cout).astype(jnp.float32),
      ad.reshape(cout, 1).astype(jnp.float32))


def _edge_body(cout, r, yp_ref, ae_ref, ypad_ref):
    s = yp_ref[0] + yp_ref[1]
    deg = s[:, cout:cout + 1]
    inv = 1.0 / jnp.maximum(deg, 1.0)
    y = s[:, :cout] * inv
    ypad_ref[:, :cout] = y
    alpha = jnp.dot(y, ae_ref[...], precision=_HIGH,
                    preferred_element_type=jnp.float32)
    lane = lax.broadcasted_iota(jnp.int32, (y.shape[0], r - cout), 1)
    pad = jnp.where(lane == 0, 1.0, jnp.where(lane == 1, alpha, 0.0))
    ypad_ref[:, cout:] = pad


def _tc_edge(yparts, ae, cout, r):
    bm = 400
    yp = yparts.reshape(2, _M, r)
    return pl.pallas_call(
        functools.partial(_edge_body, cout, r),
        grid=(_M // bm,),
        in_specs=[
            pl.BlockSpec((2, bm, r), lambda i: (0, i, 0)),
            pl.BlockSpec((cout, 1), lambda i: (0, 0)),
        ],
        out_specs=pl.BlockSpec((bm, r), lambda i: (i, 0)),
        out_shape=jax.ShapeDtypeStruct((_M, r), jnp.float32),
    )(yp, ae.reshape(cout, 1).astype(jnp.float32))


def _vert_body(cout, r, nh, *refs):
    out_ref = refs[-1]
    acc = None
    for xr in refs[:-1]:
        s = xr[0] + xr[1]
        den = s[:, cout:cout + 1] + 1e-12
        xo = s[:, :cout] / den
        e = jnp.where(xo > 0, xo, jnp.exp(jnp.minimum(xo, 0.0)) - 1.0)
        acc = e if acc is None else acc + e
    out_ref[...] = acc * (1.0 / nh)


def _tc_vertex(xoparts, cout, r):
    nh = len(xoparts)
    bn = 400
    xps = [x.reshape(2, _N, r) for x in xoparts]
    return pl.pallas_call(
        functools.partial(_vert_body, cout, r, nh),
        grid=(_N // bn,),
        in_specs=[pl.BlockSpec((2, bn, r), lambda i: (0, i, 0))
                  for _ in range(nh)],
        out_specs=pl.BlockSpec((bn, cout), lambda i: (i, 0)),
        out_shape=jax.ShapeDtypeStruct((_N, cout), jnp.float32),
    )(*xps)


_NZCHUNK = _M // _CHUNK
_ZPT = -(-_NZCHUNK // _NS)


def _zero_rowbuf(row_buf, ncg):
    @pl.loop(0, _CHUNK)
    def _(i):
        for g in range(ncg):
            row_buf[i, pl.ds(g * 16, 16)] = jnp.zeros((16,), jnp.float32)


def _zero_and_barrier(row_buf, acc_sh, s, ncg, sem):
    _zero_rowbuf(row_buf, ncg)
    for k in range(_ZPT):
        j = s + k * _NS

        @pl.when(j < _NZCHUNK)
        def _():
            pltpu.async_copy(row_buf, acc_sh.at[pl.ds(j * _CHUNK, _CHUNK)],
                             sem)
    for k in range(_ZPT):
        j = s + k * _NS

        @pl.when(j < _NZCHUNK)
        def _():
            pltpu.make_async_copy(
                row_buf, acc_sh.at[pl.ds(j * _CHUNK, _CHUNK)], sem).wait()
    plsc.subcore_barrier()


def _writeout(acc_sh, out_hbm, c, s, sem):
    plsc.subcore_barrier()
    for k in range(_ZPT):
        j = s + k * _NS

        @pl.when(j < _NZCHUNK)
        def _():
            off = j * _CHUNK
            pltpu.async_copy(acc_sh.at[pl.ds(off, _CHUNK)],
                             out_hbm.at[pl.ds(c * _M + off, _CHUNK)], sem)
    for k in range(_ZPT):
        j = s + k * _NS

        @pl.when(j < _NZCHUNK)
        def _():
            off = j * _CHUNK
            pltpu.make_async_copy(
                acc_sh.at[pl.ds(off, _CHUNK)],
                out_hbm.at[pl.ds(c * _M + off, _CHUNK)], sem).wait()


def _fetch_idx(pvpe_hbm, wid, sup, idx_buf, sem_i):
    pltpu.async_copy(pvpe_hbm.at[wid, pl.ds(sup * _SCH, _SCH)], idx_buf,
                     sem_i)


def _wait_idx(pvpe_hbm, wid, idx_buf, sem_i):
    pltpu.make_async_copy(pvpe_hbm.at[wid, pl.ds(0, _SCH)], idx_buf,
                          sem_i).wait()


def _v2e_sup(first, gather_hbm, cur, prev, row_bufs, acc_sh,
             sems, sem_g, prefetch):
    gd = {}
    for i in range(_SCH):
        b = i % 3
        if i < 3:
            if not first:
                pltpu.make_async_copy(
                    row_bufs.at[b], acc_sh.at[prev.at[_SCH - 3 + i, 1]],
                    sems[b]).wait()
        else:
            pltpu.make_async_copy(
                row_bufs.at[b], acc_sh.at[cur.at[i - 3, 1]], sems[b]).wait()
        gd[i] = pltpu.async_copy(gather_hbm.at[cur.at[i, 0]],
                                 row_bufs.at[b], sem_g)
        if i == 2 and prefetch is not None:
            prefetch()
        if i >= 1:
            gd[i - 1].wait()
            pltpu.async_copy(row_bufs.at[(i - 1) % 3],
                             acc_sh.at[cur.at[i - 1, 1]],
                             sems[(i - 1) % 3], add=True)
    gd[_SCH - 1].wait()
    pltpu.async_copy(row_bufs.at[(_SCH - 1) % 3],
                     acc_sh.at[cur.at[_SCH - 1, 1]],
                     sems[(_SCH - 1) % 3], add=True)


def _v2e_body(r, xlp_hbm, pvpe_hbm, out_hbm,
              idx_a, idx_b, row_bufs, acc_sh, sem_i, sem_g, s0, s1, s2):
    c = lax.axis_index("c")
    s = lax.axis_index("s")
    wid = c * _NS + s
    sems = (s0, s1, s2)
    _fetch_idx(pvpe_hbm, wid, 0, idx_a, sem_i)
    _zero_and_barrier(row_bufs.at[0], acc_sh, s, r // 16, sem_g)

    _wait_idx(pvpe_hbm, wid, idx_a, sem_i)
    _v2e_sup(True, xlp_hbm, idx_a, idx_b, row_bufs, acc_sh, sems, sem_g,
             lambda: _fetch_idx(pvpe_hbm, wid, 1, idx_b, sem_i))

    npair = (_NSUP - 1) // 2

    @pl.loop(0, npair)
    def _(t):
        _wait_idx(pvpe_hbm, wid, idx_b, sem_i)
        _v2e_sup(False, xlp_hbm, idx_b, idx_a, row_bufs, acc_sh,
                 sems, sem_g,
                 lambda: _fetch_idx(pvpe_hbm, wid, 2 * t + 2, idx_a, sem_i))
        _wait_idx(pvpe_hbm, wid, idx_a, sem_i)

        def pf():
            @pl.when(t < npair - 1)
            def _():
                _fetch_idx(pvpe_hbm, wid, 2 * t + 3, idx_b, sem_i)
        _v2e_sup(False, xlp_hbm, idx_a, idx_b, row_bufs, acc_sh,
                 sems, sem_g, pf)

    for i in range(3):
        b = (_SCH - 3 + i) % 3
        pltpu.make_async_copy(row_bufs.at[b],
                              acc_sh.at[idx_a.at[_SCH - 3 + i, 1]],
                              sems[b]).wait()

    _writeout(acc_sh, out_hbm, c, s, sem_g)


def _sc_v2e(xl_pad, pvpe, r):
    k = pl.kernel(
        functools.partial(_v2e_body, r),
        out_type=jax.ShapeDtypeStruct((_NC * _M, r), jnp.float32),
        mesh=_mesh,
        scratch_types=[
            pltpu.VMEM((_SCH, 2, _CHUNK), jnp.int32),
            pltpu.VMEM((_SCH, 2, _CHUNK), jnp.int32),
            pltpu.VMEM((3, _CHUNK, r), jnp.float32),
            pltpu.VMEM_SHARED((_M, r), jnp.float32),
            pltpu.SemaphoreType.DMA,
            pltpu.SemaphoreType.DMA,
            pltpu.SemaphoreType.DMA,
            pltpu.SemaphoreType.DMA,
            pltpu.SemaphoreType.DMA,
        ],
        compiler_params=_SC_PARAMS,
    )
    return k(xl_pad, pvpe)


def _e2v_compute(b, cout, ncg, idx, i, row_bufs, ex_buf, xd_buf):
    @pl.loop(0, _CHUNK // 16)
    def _(g):
        g16 = g * 16 + lax.iota(jnp.int32, 16)
        al = plsc.load_gather(row_bufs.at[b],
                              [g16, jnp.full((16,), cout + 1, jnp.int32)])
        pvv = idx[i, 0, pl.ds(g * 16, 16)]
        xd = plsc.load_gather(xd_buf, [pvv])
        sc = al + xd
        sc = jnp.where(sc >= 0, sc, _NEG * sc)
        ex_buf[pl.ds(g * 16, 16)] = jnp.exp(sc)

    @pl.loop(0, _CHUNK, unroll=4)
    def _(rr):
        exi = plsc.load_gather(ex_buf, [jnp.zeros((16,), jnp.int32) + rr])
        for g in range(ncg):
            row_bufs[b, rr, pl.ds(g * 16, 16)] = (
                row_bufs[b, rr, pl.ds(g * 16, 16)] * exi)


def _e2v_sup(first, cout, ncg, ypad_hbm, cur, prev, row_bufs, ex_buf, xd_buf,
             acc_sh, sems, sem_g, prefetch):
    gd = {}

    def process(i):
        b = i % 2
        gd[i].wait()
        _e2v_compute(b, cout, ncg, cur, i, row_bufs, ex_buf, xd_buf)
        pltpu.async_copy(row_bufs.at[b], acc_sh.at[cur.at[i, 0]],
                         sems[b], add=True)

    for i in range(_SCH):
        b = i % 2
        if i < 2:
            if not first:
                pltpu.make_async_copy(
                    row_bufs.at[b], acc_sh.at[prev.at[_SCH - 2 + i, 0]],
                    sems[b]).wait()
        else:
            pltpu.make_async_copy(
                row_bufs.at[b], acc_sh.at[cur.at[i - 2, 0]], sems[b]).wait()
        gd[i] = pltpu.async_copy(ypad_hbm.at[cur.at[i, 1]],
                                 row_bufs.at[b], sem_g)
        if i == 1 and prefetch is not None:
            prefetch()
        if i >= 1:
            process(i - 1)
    process(_SCH - 1)


def _e2v_body(cout, r, ypad_hbm, xd_hbm, pvpe_hbm, out_hbm,
              idx_a, idx_b, row_bufs, ex_buf, xd_buf, acc_sh,
              sem_i, sem_g, s0, s1):
    c = lax.axis_index("c")
    s = lax.axis_index("s")
    wid = c * _NS + s
    ncg = r // 16
    sems = (s0, s1)
    _fetch_idx(pvpe_hbm, wid, 0, idx_a, sem_i)
    pltpu.sync_copy(xd_hbm, xd_buf)
    _zero_and_barrier(row_bufs.at[0], acc_sh, s, ncg, sem_g)

    _wait_idx(pvpe_hbm, wid, idx_a, sem_i)
    _e2v_sup(True, cout, ncg, ypad_hbm, idx_a, idx_b, row_bufs, ex_buf,
             xd_buf, acc_sh, sems, sem_g,
             lambda: _fetch_idx(pvpe_hbm, wid, 1, idx_b, sem_i))

    npair = (_NSUP - 1) // 2

    @pl.loop(0, npair)
    def _(t):
        _wait_idx(pvpe_hbm, wid, idx_b, sem_i)
        _e2v_sup(False, cout, ncg, ypad_hbm, idx_b, idx_a, row_bufs, ex_buf,
                 xd_buf, acc_sh, sems, sem_g,
                 lambda: _fetch_idx(pvpe_hbm, wid, 2 * t + 2, idx_a, sem_i))
        _wait_idx(pvpe_hbm, wid, idx_a, sem_i)

        def pf():
            @pl.when(t < npair - 1)
            def _():
                _fetch_idx(pvpe_hbm, wid, 2 * t + 3, idx_b, sem_i)
        _e2v_sup(False, cout, ncg, ypad_hbm, idx_a, idx_b, row_bufs, ex_buf,
                 xd_buf, acc_sh, sems, sem_g, pf)

    for i in range(2):
        b = (_SCH - 2 + i) % 2
        pltpu.make_async_copy(row_bufs.at[b],
                              acc_sh.at[idx_a.at[_SCH - 2 + i, 0]],
                              sems[b]).wait()

    _writeout(acc_sh, out_hbm, c, s, sem_g)


def _sc_e2v(y_pad, xdst, pvpe, cout, r):
    k = pl.kernel(
        functools.partial(_e2v_body, cout, r),
        out_type=jax.ShapeDtypeStruct((_NC * _N, r), jnp.float32),
        mesh=_mesh,
        scratch_types=[
            pltpu.VMEM((_SCH, 2, _CHUNK), jnp.int32),
            pltpu.VMEM((_SCH, 2, _CHUNK), jnp.int32),
            pltpu.VMEM((2, _CHUNK, r), jnp.float32),
            pltpu.VMEM((_CHUNK,), jnp.float32),
            pltpu.VMEM((_N,), jnp.float32),
            pltpu.VMEM_SHARED((_N, r), jnp.float32),
            pltpu.SemaphoreType.DMA,
            pltpu.SemaphoreType.DMA,
            pltpu.SemaphoreType.DMA,
            pltpu.SemaphoreType.DMA,
        ],
        compiler_params=_SC_PARAMS,
    )
    return k(y_pad, xdst, pvpe)


def _conv(h, pvpe, w, b, ae, ad, cout):
    r = cout + 16
    xl_pad, xdst = _tc_linear(h, w, b, ad, cout, r)
    yparts = _sc_v2e(xl_pad, pvpe, r)
    y_pad = _tc_edge(yparts, ae, cout, r)
    return _sc_e2v(y_pad, xdst.reshape(_N), pvpe, cout, r)


def kernel(X, pair_v, pair_e, W_layers, b_layers, ae_layers, ad_layers,
           W_out, b_out, ae_out, ad_out):
    h = X.astype(jnp.float32)
    pv3 = pair_v.astype(jnp.int32).reshape(_NW, _NCHUNK, 1, _CHUNK)
    pe3 = pair_e.astype(jnp.int32).reshape(_NW, _NCHUNK, 1, _CHUNK)
    pvpe = jnp.concatenate([pv3, pe3], axis=2)
    for l in range(_DEPTH):
        parts = [
            _conv(h, pvpe, W_layers[l, hd], b_layers[l, hd],
                  ae_layers[l, hd], ad_layers[l, hd], _C)
            for hd in range(_HEADS)
        ]
        h = _tc_vertex(parts, _C, _C + 16)
    xop = _conv(h, pvpe, W_out, b_out, ae_out, ad_out, _K)
    return _tc_vertex([xop], _K, _K + 16)


# speedup vs baseline: 1.6940x; 1.0102x over previous
import functools

import jax
import jax.numpy as jnp
from jax import lax
from jax.experimental import pallas as pl
from jax.experimental.pallas import tpu as pltpu
from jax.experimental.pallas import tpu_sc as plsc

_N = 10000
_M = 10000
_P = 320000
_C = 128
_K = 64
_DEPTH = 2
_HEADS = 4
_NEG = 0.2

_NC = 2
_NS = 16
_NW = _NC * _NS
_CHUNK = 80
_PPT = _P // _NW
_NCHUNK = _PPT // _CHUNK
_SCH = 5
_NSUP = _NCHUNK // _SCH

_HIGH = lax.Precision.HIGHEST

_SC_PARAMS = pltpu.CompilerParams(use_tc_tiling_on_sc=False,
                                  needs_layout_passes=False)

_mesh = plsc.VectorSubcoreMesh(core_axis_name="c", subcore_axis_name="s",
                               num_cores=_NC, num_subcores=_NS)


def _lin_body(cout, r, h_ref, w_ref, b_ref, ad_ref, xlp_ref, xd_ref):
    xl = jnp.dot(h_ref[...], w_ref[...], precision=_HIGH,
                 preferred_element_type=jnp.float32) + b_ref[...]
    xlp_ref[:, :cout] = xl
    pad = jnp.where(
        lax.broadcasted_iota(jnp.int32, (xl.shape[0], r - cout), 1) == 0,
        1.0, 0.0)
    xlp_ref[:, cout:] = pad
    xd_ref[...] = jnp.dot(xl, ad_ref[...], precision=_HIGH,
                          preferred_element_type=jnp.float32)


def _tc_linear(h, w, b, ad, cout, r):
    n, cin = h.shape
    bn = 400
    return pl.pallas_call(
        functools.partial(_lin_body, cout, r),
        grid=(n // bn,),
        in_specs=[
            pl.BlockSpec((bn, cin), lambda i: (i, 0)),
            pl.BlockSpec((cin, cout), lambda i: (0, 0)),
            pl.BlockSpec((1, cout), lambda i: (0, 0)),
            pl.BlockSpec((cout, 1), lambda i: (0, 0)),
        ],
        out_specs=[
            pl.BlockSpec((bn, r), lambda i: (i, 0)),
            pl.BlockSpec((bn, 1), lambda i: (i, 0)),
        ],
        out_shape=[
            jax.ShapeDtypeStruct((n, r), jnp.float32),
            jax.ShapeDtypeStruct((n, 1), jnp.float32),
        ],
    )(h, w.astype(jnp.float32), b.reshape(1, cout).astype(jnp.float32),
      ad.reshape(cout, 1).astype(jnp.float32))


def _edge_body(cout, r, yp_ref, ae_ref, ypad_ref):
    s = yp_ref[0] + yp_ref[1]
    deg = s[:, cout:cout + 1]
    inv = 1.0 / jnp.maximum(deg, 1.0)
    y = s[:, :cout] * inv
    ypad_ref[:, :cout] = y
    alpha = jnp.dot(y, ae_ref[...], precision=_HIGH,
                    preferred_element_type=jnp.float32)
    lane = lax.broadcasted_iota(jnp.int32, (y.shape[0], r - cout), 1)
    pad = jnp.where(lane == 0, 1.0, jnp.where(lane == 1, alpha, 0.0))
    ypad_ref[:, cout:] = pad


def _tc_edge(yparts, ae, cout, r):
    bm = 400
    yp = yparts.reshape(2, _M, r)
    return pl.pallas_call(
        functools.partial(_edge_body, cout, r),
        grid=(_M // bm,),
        in_specs=[
            pl.BlockSpec((2, bm, r), lambda i: (0, i, 0)),
            pl.BlockSpec((cout, 1), lambda i: (0, 0)),
        ],
        out_specs=pl.BlockSpec((bm, r), lambda i: (i, 0)),
        out_shape=jax.ShapeDtypeStruct((_M, r), jnp.float32),
    )(yp, ae.reshape(cout, 1).astype(jnp.float32))


def _vert_body(cout, r, nh, *refs):
    out_ref = refs[-1]
    acc = None
    for xr in refs[:-1]:
        s = xr[0] + xr[1]
        den = s[:, cout:cout + 1] + 1e-12
        xo = s[:, :cout] / den
        e = jnp.where(xo > 0, xo, jnp.exp(jnp.minimum(xo, 0.0)) - 1.0)
        acc = e if acc is None else acc + e
    out_ref[...] = acc * (1.0 / nh)


def _tc_vertex(xoparts, cout, r):
    nh = len(xoparts)
    bn = 400
    xps = [x.reshape(2, _N, r) for x in xoparts]
    return pl.pallas_call(
        functools.partial(_vert_body, cout, r, nh),
        grid=(_N // bn,),
        in_specs=[pl.BlockSpec((2, bn, r), lambda i: (0, i, 0))
                  for _ in range(nh)],
        out_specs=pl.BlockSpec((bn, cout), lambda i: (i, 0)),
        out_shape=jax.ShapeDtypeStruct((_N, cout), jnp.float32),
    )(*xps)


_NZCHUNK = _M // _CHUNK
_ZPT = -(-_NZCHUNK // _NS)


def _zero_rowbuf(row_buf, nrow, ncg):
    @pl.loop(0, nrow)
    def _(i):
        for g in range(ncg):
            row_buf[i, pl.ds(g * 16, 16)] = jnp.zeros((16,), jnp.float32)


def _zero_and_barrier(row_buf, acc_sh, s, nrow, ncg):
    _zero_rowbuf(row_buf, nrow, ncg)

    @pl.loop(0, _ZPT)
    def _(k):
        j = s + k * _NS

        @pl.when(j < _NZCHUNK)
        def _():
            pltpu.sync_copy(row_buf, acc_sh.at[pl.ds(j * nrow, nrow)])

    plsc.subcore_barrier()


def _writeout(acc_sh, out_hbm, c, s, nrow, r):
    plsc.subcore_barrier()

    @pl.loop(0, _ZPT)
    def _(k):
        j = s + k * _NS

        @pl.when(j < _NZCHUNK)
        def _():
            off = j * nrow
            pltpu.sync_copy(acc_sh.at[pl.ds(off, nrow)],
                            out_hbm.at[pl.ds(c * _M + off, nrow)])


_SCH15 = 15
_NFULL = _NCHUNK // _SCH15
_TAIL = _NCHUNK % _SCH15


def _drain_all(nbuf, row_bufs, acc_sh, idx_buf, sems):
    for b in range(nbuf):
        pltpu.make_async_copy(row_bufs.at[b], acc_sh.at[idx_buf.at[0, 1]],
                              sems[b]).wait()


def _v2e_steps(nsteps, xlp_hbm, idx_buf, row_bufs, acc_sh, sems, sem_g):
    gd = {}
    for i in range(nsteps):
        b = i % 3
        if i >= 3:
            pltpu.make_async_copy(row_bufs.at[b],
                                  acc_sh.at[idx_buf.at[i - 3, 1]],
                                  sems[b]).wait()
        gd[i] = pltpu.async_copy(xlp_hbm.at[idx_buf.at[i, 0]],
                                 row_bufs.at[b], sem_g)
        if i >= 1:
            gd[i - 1].wait()
            pltpu.async_copy(row_bufs.at[(i - 1) % 3],
                             acc_sh.at[idx_buf.at[i - 1, 1]],
                             sems[(i - 1) % 3], add=True)
    gd[nsteps - 1].wait()
    pltpu.async_copy(row_bufs.at[(nsteps - 1) % 3],
                     acc_sh.at[idx_buf.at[nsteps - 1, 1]],
                     sems[(nsteps - 1) % 3], add=True)


def _v2e_body(r, xlp_hbm, pvpe_hbm, out_hbm,
              idx_buf, row_bufs, acc_sh, sem_i, sem_g, s0, s1, s2):
    c = lax.axis_index("c")
    s = lax.axis_index("s")
    wid = c * _NS + s
    sems = (s0, s1, s2)
    _zero_and_barrier(row_bufs.at[0], acc_sh, s, _CHUNK, r // 16)

    @pl.loop(0, _NFULL)
    def _(j):
        @pl.when(j > 0)
        def _():
            _drain_all(3, row_bufs, acc_sh, idx_buf, sems)

        pltpu.async_copy(pvpe_hbm.at[wid, pl.ds(j * _SCH15, _SCH15)],
                         idx_buf, sem_i).wait()
        _v2e_steps(_SCH15, xlp_hbm, idx_buf, row_bufs, acc_sh, sems, sem_g)

    if _TAIL:
        _drain_all(3, row_bufs, acc_sh, idx_buf, sems)
        pltpu.async_copy(pvpe_hbm.at[wid, pl.ds(_NFULL * _SCH15, _TAIL)],
                         idx_buf.at[pl.ds(0, _TAIL)], sem_i).wait()
        _v2e_steps(_TAIL, xlp_hbm, idx_buf, row_bufs, acc_sh, sems, sem_g)

    _drain_all(min(3, _TAIL if _TAIL else _SCH15), row_bufs, acc_sh,
               idx_buf, sems)
    _writeout(acc_sh, out_hbm, c, s, _CHUNK, r)


def _sc_v2e(xl_pad, pvpe, r):
    k = pl.kernel(
        functools.partial(_v2e_body, r),
        out_type=jax.ShapeDtypeStruct((_NC * _M, r), jnp.float32),
        mesh=_mesh,
        scratch_types=[
            pltpu.VMEM((_SCH15, 2, _CHUNK), jnp.int32),
            pltpu.VMEM((3, _CHUNK, r), jnp.float32),
            pltpu.VMEM_SHARED((_M, r), jnp.float32),
            pltpu.SemaphoreType.DMA,
            pltpu.SemaphoreType.DMA,
            pltpu.SemaphoreType.DMA,
            pltpu.SemaphoreType.DMA,
            pltpu.SemaphoreType.DMA,
        ],
        compiler_params=_SC_PARAMS,
    )
    return k(xl_pad, pvpe)


def _e2v_compute(b, cout, ncg, idx_buf, i, row_bufs, ex_buf, xd_buf):
    @pl.loop(0, _CHUNK // 16)
    def _(g):
        g16 = g * 16 + lax.iota(jnp.int32, 16)
        al = plsc.load_gather(row_bufs.at[b],
                              [g16, jnp.full((16,), cout + 1, jnp.int32)])
        pvv = idx_buf[i, 0, pl.ds(g * 16, 16)]
        xd = plsc.load_gather(xd_buf, [pvv])
        sc = al + xd
        sc = jnp.where(sc >= 0, sc, _NEG * sc)
        ex_buf[pl.ds(g * 16, 16)] = jnp.exp(sc)

    @pl.loop(0, _CHUNK, unroll=4)
    def _(rr):
        exi = plsc.load_gather(ex_buf, [jnp.zeros((16,), jnp.int32) + rr])
        for g in range(ncg):
            row_bufs[b, rr, pl.ds(g * 16, 16)] = (
                row_bufs[b, rr, pl.ds(g * 16, 16)] * exi)


def _e2v_steps(nsteps, cout, ncg, ypad_hbm, idx_buf, row_bufs,
               ex_buf, xd_buf, acc_sh, sems, sem_g):
    gd = {}

    def process(i):
        b = i % 2
        gd[i].wait()
        _e2v_compute(b, cout, ncg, idx_buf, i, row_bufs, ex_buf, xd_buf)
        pltpu.async_copy(row_bufs.at[b], acc_sh.at[idx_buf.at[i, 0]],
                         sems[b], add=True)

    for i in range(nsteps):
        b = i % 2
        if i >= 2:
            pltpu.make_async_copy(row_bufs.at[b],
                                  acc_sh.at[idx_buf.at[i - 2, 0]],
                                  sems[b]).wait()
        gd[i] = pltpu.async_copy(ypad_hbm.at[idx_buf.at[i, 1]],
                                 row_bufs.at[b], sem_g)
        if i >= 1:
            process(i - 1)
    process(nsteps - 1)


def _e2v_body(cout, r, ypad_hbm, xd_hbm, pvpe_hbm, out_hbm,
              idx_buf, row_bufs, ex_buf, xd_buf, acc_sh,
              sem_i, sem_g, s0, s1):
    c = lax.axis_index("c")
    s = lax.axis_index("s")
    wid = c * _NS + s
    ncg = r // 16
    sems = (s0, s1)
    pltpu.sync_copy(xd_hbm, xd_buf)
    _zero_and_barrier(row_bufs.at[0], acc_sh, s, _CHUNK, ncg)

    @pl.loop(0, _NFULL)
    def _(j):
        @pl.when(j > 0)
        def _():
            _drain_all(2, row_bufs, acc_sh, idx_buf, sems)

        pltpu.async_copy(pvpe_hbm.at[wid, pl.ds(j * _SCH15, _SCH15)],
                         idx_buf, sem_i).wait()
        _e2v_steps(_SCH15, cout, ncg, ypad_hbm, idx_buf, row_bufs,
                   ex_buf, xd_buf, acc_sh, sems, sem_g)

    if _TAIL:
        _drain_all(2, row_bufs, acc_sh, idx_buf, sems)
        pltpu.async_copy(pvpe_hbm.at[wid, pl.ds(_NFULL * _SCH15, _TAIL)],
                         idx_buf.at[pl.ds(0, _TAIL)], sem_i).wait()
        _e2v_steps(_TAIL, cout, ncg, ypad_hbm, idx_buf, row_bufs,
                   ex_buf, xd_buf, acc_sh, sems, sem_g)

    _drain_all(min(2, _TAIL if _TAIL else _SCH15), row_bufs, acc_sh,
               idx_buf, sems)
    _writeout(acc_sh, out_hbm, c, s, _CHUNK, r)


def _sc_e2v(y_pad, xdst, pvpe, cout, r):
    k = pl.kernel(
        functools.partial(_e2v_body, cout, r),
        out_type=jax.ShapeDtypeStruct((_NC * _N, r), jnp.float32),
        mesh=_mesh,
        scratch_types=[
            pltpu.VMEM((_SCH15, 2, _CHUNK), jnp.int32),
            pltpu.VMEM((2, _CHUNK, r), jnp.float32),
            pltpu.VMEM((_CHUNK,), jnp.float32),
            pltpu.VMEM((_N,), jnp.float32),
            pltpu.VMEM_SHARED((_N, r), jnp.float32),
            pltpu.SemaphoreType.DMA,
            pltpu.SemaphoreType.DMA,
            pltpu.SemaphoreType.DMA,
            pltpu.SemaphoreType.DMA,
        ],
        compiler_params=_SC_PARAMS,
    )
    return k(y_pad, xdst, pvpe)


def _conv(h, pvpe, w, b, ae, ad, cout):
    r = cout + 16
    xl_pad, xdst = _tc_linear(h, w, b, ad, cout, r)
    yparts = _sc_v2e(xl_pad, pvpe, r)
    y_pad = _tc_edge(yparts, ae, cout, r)
    return _sc_e2v(y_pad, xdst.reshape(_N), pvpe, cout, r)


def kernel(X, pair_v, pair_e, W_layers, b_layers, ae_layers, ad_layers,
           W_out, b_out, ae_out, ad_out):
    h = X.astype(jnp.float32)
    pv3 = pair_v.astype(jnp.int32).reshape(_NW, _NCHUNK, 1, _CHUNK)
    pe3 = pair_e.astype(jnp.int32).reshape(_NW, _NCHUNK, 1, _CHUNK)
    pvpe = jnp.concatenate([pv3, pe3], axis=2)
    for l in range(_DEPTH):
        parts = [
            _conv(h, pvpe, W_layers[l, hd], b_layers[l, hd],
                  ae_layers[l, hd], ad_layers[l, hd], _C)
            for hd in range(_HEADS)
        ]
        h = _tc_vertex(parts, _C, _C + 16)
    xop = _conv(h, pvpe, W_out, b_out, ae_out, ad_out, _K)
    return _tc_vertex([xop], _K, _K + 16)


# speedup vs baseline: 1.7383x; 1.0262x over previous
import functools

import jax
import jax.numpy as jnp
from jax import lax
from jax.experimental import pallas as pl
from jax.experimental.pallas import tpu as pltpu
from jax.experimental.pallas import tpu_sc as plsc

_N = 10000
_M = 10000
_P = 320000
_C = 128
_K = 64
_DEPTH = 2
_HEADS = 4
_NEG = 0.2

_NC = 2
_NS = 16
_NW = _NC * _NS
_CHUNK = 80
_PPT = _P // _NW
_NCHUNK = _PPT // _CHUNK
_SCH = 5
_NSUP = _NCHUNK // _SCH

_HIGH = lax.Precision.HIGHEST

_SC_PARAMS = pltpu.CompilerParams(use_tc_tiling_on_sc=False,
                                  needs_layout_passes=False)

_mesh = plsc.VectorSubcoreMesh(core_axis_name="c", subcore_axis_name="s",
                               num_cores=_NC, num_subcores=_NS)


def _lin_body(cout, r, h_ref, w_ref, b_ref, ad_ref, xlp_ref, xd_ref):
    xl = jnp.dot(h_ref[...], w_ref[...], precision=_HIGH,
                 preferred_element_type=jnp.float32) + b_ref[...]
    xlp_ref[:, :cout] = xl
    pad = jnp.where(
        lax.broadcasted_iota(jnp.int32, (xl.shape[0], r - cout), 1) == 0,
        1.0, 0.0)
    xlp_ref[:, cout:] = pad
    xd_ref[...] = jnp.dot(xl, ad_ref[...], precision=_HIGH,
                          preferred_element_type=jnp.float32)


def _tc_linear(h, w, b, ad, cout, r):
    n, cin = h.shape
    bn = 400
    return pl.pallas_call(
        functools.partial(_lin_body, cout, r),
        grid=(n // bn,),
        in_specs=[
            pl.BlockSpec((bn, cin), lambda i: (i, 0)),
            pl.BlockSpec((cin, cout), lambda i: (0, 0)),
            pl.BlockSpec((1, cout), lambda i: (0, 0)),
            pl.BlockSpec((cout, 1), lambda i: (0, 0)),
        ],
        out_specs=[
            pl.BlockSpec((bn, r), lambda i: (i, 0)),
            pl.BlockSpec((bn, 1), lambda i: (i, 0)),
        ],
        out_shape=[
            jax.ShapeDtypeStruct((n, r), jnp.float32),
            jax.ShapeDtypeStruct((n, 1), jnp.float32),
        ],
    )(h, w.astype(jnp.float32), b.reshape(1, cout).astype(jnp.float32),
      ad.reshape(cout, 1).astype(jnp.float32))


def _edge_body(cout, r, yp_ref, ae_ref, ypad_ref):
    s = yp_ref[0] + yp_ref[1]
    deg = s[:, cout:cout + 1]
    inv = 1.0 / jnp.maximum(deg, 1.0)
    y = s[:, :cout] * inv
    ypad_ref[:, :cout] = y
    alpha = jnp.dot(y, ae_ref[...], precision=_HIGH,
                    preferred_element_type=jnp.float32)
    lane = lax.broadcasted_iota(jnp.int32, (y.shape[0], r - cout), 1)
    pad = jnp.where(lane == 0, 1.0, jnp.where(lane == 1, alpha, 0.0))
    ypad_ref[:, cout:] = pad


def _tc_edge(yparts, ae, cout, r):
    bm = 400
    yp = yparts.reshape(2, _M, r)
    return pl.pallas_call(
        functools.partial(_edge_body, cout, r),
        grid=(_M // bm,),
        in_specs=[
            pl.BlockSpec((2, bm, r), lambda i: (0, i, 0)),
            pl.BlockSpec((cout, 1), lambda i: (0, 0)),
        ],
        out_specs=pl.BlockSpec((bm, r), lambda i: (i, 0)),
        out_shape=jax.ShapeDtypeStruct((_M, r), jnp.float32),
    )(yp, ae.reshape(cout, 1).astype(jnp.float32))


def _vert_body(cout, r, nh, *refs):
    out_ref = refs[-1]
    acc = None
    for xr in refs[:-1]:
        s = xr[0] + xr[1]
        den = s[:, cout:cout + 1] + 1e-12
        xo = s[:, :cout] / den
        e = jnp.where(xo > 0, xo, jnp.exp(jnp.minimum(xo, 0.0)) - 1.0)
        acc = e if acc is None else acc + e
    out_ref[...] = acc * (1.0 / nh)


def _tc_vertex(xoparts, cout, r):
    nh = len(xoparts)
    bn = 400
    xps = [x.reshape(2, _N, r) for x in xoparts]
    return pl.pallas_call(
        functools.partial(_vert_body, cout, r, nh),
        grid=(_N // bn,),
        in_specs=[pl.BlockSpec((2, bn, r), lambda i: (0, i, 0))
                  for _ in range(nh)],
        out_specs=pl.BlockSpec((bn, cout), lambda i: (i, 0)),
        out_shape=jax.ShapeDtypeStruct((_N, cout), jnp.float32),
    )(*xps)


_NZCHUNK = _M // _CHUNK
_ZPT = -(-_NZCHUNK // _NS)


def _zero_rowbuf(row_buf, nrow, ncg):
    @pl.loop(0, nrow)
    def _(i):
        for g in range(ncg):
            row_buf[i, pl.ds(g * 16, 16)] = jnp.zeros((16,), jnp.float32)


def _zero_and_barrier(row_buf, acc_sh, s, nrow, ncg):
    _zero_rowbuf(row_buf, nrow, ncg)

    @pl.loop(0, _ZPT)
    def _(k):
        j = s + k * _NS

        @pl.when(j < _NZCHUNK)
        def _():
            pltpu.sync_copy(row_buf, acc_sh.at[pl.ds(j * nrow, nrow)])

    plsc.subcore_barrier()


def _writeout(acc_sh, out_hbm, c, s, nrow, r):
    plsc.subcore_barrier()

    @pl.loop(0, _ZPT)
    def _(k):
        j = s + k * _NS

        @pl.when(j < _NZCHUNK)
        def _():
            off = j * nrow
            pltpu.sync_copy(acc_sh.at[pl.ds(off, nrow)],
                            out_hbm.at[pl.ds(c * _M + off, nrow)])


_SCH15 = 25
_NFULL = _NCHUNK // _SCH15
_TAIL = _NCHUNK % _SCH15


def _drain_all(nbuf, row_bufs, acc_sh, idx_buf, sems):
    for b in range(nbuf):
        pltpu.make_async_copy(row_bufs.at[b], acc_sh.at[idx_buf.at[0, 1]],
                              sems[b]).wait()


def _v2e_steps(nsteps, xlp_hbm, idx_buf, row_bufs, acc_sh, sems, sem_g):
    gd = {}
    for i in range(nsteps):
        b = i % 3
        if i >= 3:
            pltpu.make_async_copy(row_bufs.at[b],
                                  acc_sh.at[idx_buf.at[i - 3, 1]],
                                  sems[b]).wait()
        gd[i] = pltpu.async_copy(xlp_hbm.at[idx_buf.at[i, 0]],
                                 row_bufs.at[b], sem_g)
        if i >= 1:
            gd[i - 1].wait()
            pltpu.async_copy(row_bufs.at[(i - 1) % 3],
                             acc_sh.at[idx_buf.at[i - 1, 1]],
                             sems[(i - 1) % 3], add=True)
    gd[nsteps - 1].wait()
    pltpu.async_copy(row_bufs.at[(nsteps - 1) % 3],
                     acc_sh.at[idx_buf.at[nsteps - 1, 1]],
                     sems[(nsteps - 1) % 3], add=True)


def _v2e_body(r, xlp_hbm, pvpe_hbm, out_hbm,
              idx_buf, row_bufs, acc_sh, sem_i, sem_g, s0, s1, s2):
    c = lax.axis_index("c")
    s = lax.axis_index("s")
    wid = c * _NS + s
    sems = (s0, s1, s2)
    _zero_and_barrier(row_bufs.at[0], acc_sh, s, _CHUNK, r // 16)

    @pl.loop(0, _NFULL)
    def _(j):
        @pl.when(j > 0)
        def _():
            _drain_all(3, row_bufs, acc_sh, idx_buf, sems)

        pltpu.async_copy(pvpe_hbm.at[wid, pl.ds(j * _SCH15, _SCH15)],
                         idx_buf, sem_i).wait()
        _v2e_steps(_SCH15, xlp_hbm, idx_buf, row_bufs, acc_sh, sems, sem_g)

    if _TAIL:
        _drain_all(3, row_bufs, acc_sh, idx_buf, sems)
        pltpu.async_copy(pvpe_hbm.at[wid, pl.ds(_NFULL * _SCH15, _TAIL)],
                         idx_buf.at[pl.ds(0, _TAIL)], sem_i).wait()
        _v2e_steps(_TAIL, xlp_hbm, idx_buf, row_bufs, acc_sh, sems, sem_g)

    _drain_all(min(3, _TAIL if _TAIL else _SCH15), row_bufs, acc_sh,
               idx_buf, sems)
    _writeout(acc_sh, out_hbm, c, s, _CHUNK, r)


def _sc_v2e(xl_pad, pvpe, r):
    k = pl.kernel(
        functools.partial(_v2e_body, r),
        out_type=jax.ShapeDtypeStruct((_NC * _M, r), jnp.float32),
        mesh=_mesh,
        scratch_types=[
            pltpu.VMEM((_SCH15, 2, _CHUNK), jnp.int32),
            pltpu.VMEM((3, _CHUNK, r), jnp.float32),
            pltpu.VMEM_SHARED((_M, r), jnp.float32),
            pltpu.SemaphoreType.DMA,
            pltpu.SemaphoreType.DMA,
            pltpu.SemaphoreType.DMA,
            pltpu.SemaphoreType.DMA,
            pltpu.SemaphoreType.DMA,
        ],
        compiler_params=_SC_PARAMS,
    )
    return k(xl_pad, pvpe)


def _e2v_compute(b, cout, ncg, idx_buf, i, row_bufs, ex_buf, xd_buf):
    @pl.loop(0, _CHUNK // 16)
    def _(g):
        g16 = g * 16 + lax.iota(jnp.int32, 16)
        al = plsc.load_gather(row_bufs.at[b],
                              [g16, jnp.full((16,), cout + 1, jnp.int32)])
        pvv = idx_buf[i, 0, pl.ds(g * 16, 16)]
        xd = plsc.load_gather(xd_buf, [pvv])
        sc = al + xd
        sc = jnp.where(sc >= 0, sc, _NEG * sc)
        ex_buf[pl.ds(g * 16, 16)] = jnp.exp(sc)

    @pl.loop(0, _CHUNK, unroll=4)
    def _(rr):
        exi = plsc.load_gather(ex_buf, [jnp.zeros((16,), jnp.int32) + rr])
        for g in range(ncg):
            row_bufs[b, rr, pl.ds(g * 16, 16)] = (
                row_bufs[b, rr, pl.ds(g * 16, 16)] * exi)


def _e2v_steps(nsteps, cout, ncg, ypad_hbm, idx_buf, row_bufs,
               ex_buf, xd_buf, acc_sh, sems, sem_g):
    gd = {}

    def process(i):
        b = i % 2
        gd[i].wait()
        _e2v_compute(b, cout, ncg, idx_buf, i, row_bufs, ex_buf, xd_buf)
        pltpu.async_copy(row_bufs.at[b], acc_sh.at[idx_buf.at[i, 0]],
                         sems[b], add=True)

    for i in range(nsteps):
        b = i % 2
        if i >= 2:
            pltpu.make_async_copy(row_bufs.at[b],
                                  acc_sh.at[idx_buf.at[i - 2, 0]],
                                  sems[b]).wait()
        gd[i] = pltpu.async_copy(ypad_hbm.at[idx_buf.at[i, 1]],
                                 row_bufs.at[b], sem_g)
        if i >= 1:
            process(i - 1)
    process(nsteps - 1)


def _e2v_body(cout, r, ypad_hbm, xd_hbm, pvpe_hbm, out_hbm,
              idx_buf, row_bufs, ex_buf, xd_buf, acc_sh,
              sem_i, sem_g, s0, s1):
    c = lax.axis_index("c")
    s = lax.axis_index("s")
    wid = c * _NS + s
    ncg = r // 16
    sems = (s0, s1)
    pltpu.sync_copy(xd_hbm, xd_buf)
    _zero_and_barrier(row_bufs.at[0], acc_sh, s, _CHUNK, ncg)

    @pl.loop(0, _NFULL)
    def _(j):
        @pl.when(j > 0)
        def _():
            _drain_all(2, row_bufs, acc_sh, idx_buf, sems)

        pltpu.async_copy(pvpe_hbm.at[wid, pl.ds(j * _SCH15, _SCH15)],
                         idx_buf, sem_i).wait()
        _e2v_steps(_SCH15, cout, ncg, ypad_hbm, idx_buf, row_bufs,
                   ex_buf, xd_buf, acc_sh, sems, sem_g)

    if _TAIL:
        _drain_all(2, row_bufs, acc_sh, idx_buf, sems)
        pltpu.async_copy(pvpe_hbm.at[wid, pl.ds(_NFULL * _SCH15, _TAIL)],
                         idx_buf.at[pl.ds(0, _TAIL)], sem_i).wait()
        _e2v_steps(_TAIL, cout, ncg, ypad_hbm, idx_buf, row_bufs,
                   ex_buf, xd_buf, acc_sh, sems, sem_g)

    _drain_all(min(2, _TAIL if _TAIL else _SCH15), row_bufs, acc_sh,
               idx_buf, sems)
    _writeout(acc_sh, out_hbm, c, s, _CHUNK, r)


def _sc_e2v(y_pad, xdst, pvpe, cout, r):
    k = pl.kernel(
        functools.partial(_e2v_body, cout, r),
        out_type=jax.ShapeDtypeStruct((_NC * _N, r), jnp.float32),
        mesh=_mesh,
        scratch_types=[
            pltpu.VMEM((_SCH15, 2, _CHUNK), jnp.int32),
            pltpu.VMEM((2, _CHUNK, r), jnp.float32),
            pltpu.VMEM((_CHUNK,), jnp.float32),
            pltpu.VMEM((_N,), jnp.float32),
            pltpu.VMEM_SHARED((_N, r), jnp.float32),
            pltpu.SemaphoreType.DMA,
            pltpu.SemaphoreType.DMA,
            pltpu.SemaphoreType.DMA,
            pltpu.SemaphoreType.DMA,
        ],
        compiler_params=_SC_PARAMS,
    )
    return k(y_pad, xdst, pvpe)


def _conv(h, pvpe, w, b, ae, ad, cout):
    r = cout + 16
    xl_pad, xdst = _tc_linear(h, w, b, ad, cout, r)
    yparts = _sc_v2e(xl_pad, pvpe, r)
    y_pad = _tc_edge(yparts, ae, cout, r)
    return _sc_e2v(y_pad, xdst.reshape(_N), pvpe, cout, r)


def kernel(X, pair_v, pair_e, W_layers, b_layers, ae_layers, ad_layers,
           W_out, b_out, ae_out, ad_out):
    h = X.astype(jnp.float32)
    pv3 = pair_v.astype(jnp.int32).reshape(_NW, _NCHUNK, 1, _CHUNK)
    pe3 = pair_e.astype(jnp.int32).reshape(_NW, _NCHUNK, 1, _CHUNK)
    pvpe = jnp.concatenate([pv3, pe3], axis=2)
    for l in range(_DEPTH):
        parts = [
            _conv(h, pvpe, W_layers[l, hd], b_layers[l, hd],
                  ae_layers[l, hd], ad_layers[l, hd], _C)
            for hd in range(_HEADS)
        ]
        h = _tc_vertex(parts, _C, _C + 16)
    xop = _conv(h, pvpe, W_out, b_out, ae_out, ad_out, _K)
    return _tc_vertex([xop], _K, _K + 16)
